# Initial kernel scaffold; baseline (speedup 1.0000x reference)
import jax
import jax.numpy as jnp
from jax.experimental import pallas as pl


def kernel(x, edge_index, batch, W1, b1, W2, b2, W3, b3, Wl, bl):
    raise NotImplementedError("write your pallas kernel here")


# speedup vs baseline: 31.2687x; 31.2687x over previous
import functools

import jax
import jax.numpy as jnp
from jax import lax
from jax.experimental import pallas as pl
from jax.experimental.pallas import tpu as pltpu
from jax.experimental.pallas import tpu_sc as plsc

_N = 10000
_E = 320000
_DIN = 128
_H = 32
_DOUT = 10
_G = 64

_NC = 2
_NS = 16
_NW = _NC * _NS
_C = 128
_K = 79
_PW = _K * _C
_EP = _NW * _PW
_NPAD = 10112
_NPS = _NPAD // _NS

_mesh = plsc.VectorSubcoreMesh(
    core_axis_name="c", subcore_axis_name="s", num_cores=_NC, num_subcores=_NS)


def _deg_body(col_hbm, out_hbm, colv, hist):
  cid = lax.axis_index("c")
  sid = lax.axis_index("s")
  wid = sid * _NC + cid
  pltpu.sync_copy(col_hbm.at[wid], colv)
  zeros16 = jnp.zeros((16,), jnp.float32)
  ones16 = jnp.ones((16,), jnp.float32)

  def zero_body(i, _):
    hist[pl.ds(i * 16, 16)] = zeros16
    return 0
  lax.fori_loop(0, _NPAD // 16, zero_body, 0)

  def acc_body(t, _):
    j = t // (_C // 16)
    k = t % (_C // 16)
    idx = colv[j, pl.ds(k * 16, 16)]
    plsc.addupdate_scatter(hist, [idx], ones16)
    return 0
  lax.fori_loop(0, _K * (_C // 16), acc_body, 0)

  pltpu.sync_copy(hist, out_hbm.at[wid])


_deg_call = functools.partial(
    pl.kernel,
    out_type=jax.ShapeDtypeStruct((_NW, _NPAD), jnp.float32),
    mesh=_mesh,
    scratch_types=[
        pltpu.VMEM((_K, _C), jnp.int32),
        pltpu.VMEM((_NPAD,), jnp.float32),
    ],
    compiler_params=pltpu.CompilerParams(needs_layout_passes=False),
)(_deg_body)


def _edge_body(z_hbm, row_hbm, col_hbm, out_hbm,
               rowv, colv, g0, g1, zstage, acc, sem0, sem1):
  cid = lax.axis_index("c")
  sid = lax.axis_index("s")
  wid = sid * _NC + cid
  zeros16 = jnp.zeros((16,), jnp.float32)

  def zero_body(i, _):
    zstage[i, pl.ds(0, 16)] = zeros16
    zstage[i, pl.ds(16, 16)] = zeros16
    return 0
  lax.fori_loop(0, _NPS, zero_body, 0)
  pltpu.sync_copy(zstage, acc.at[pl.ds(sid * _NPS, _NPS)])
  pltpu.sync_copy(row_hbm.at[wid], rowv)
  pltpu.sync_copy(col_hbm.at[wid], colv)
  plsc.subcore_barrier()

  pltpu.async_copy(z_hbm.at[rowv.at[0]], g0, sem0)

  def pair_body(i, _):
    j = i * 2
    pltpu.async_copy(z_hbm.at[rowv.at[j + 1]], g1, sem1)
    pltpu.make_async_copy(z_hbm.at[rowv.at[j]], g0, sem0).wait()
    pltpu.sync_copy(g0, acc.at[colv.at[j]], add=True)
    pltpu.async_copy(z_hbm.at[rowv.at[j + 2]], g0, sem0)
    pltpu.make_async_copy(z_hbm.at[rowv.at[j + 1]], g1, sem1).wait()
    pltpu.sync_copy(g1, acc.at[colv.at[j + 1]], add=True)
    return 0
  lax.fori_loop(0, (_K - 1) // 2, pair_body, 0)

  pltpu.make_async_copy(z_hbm.at[rowv.at[_K - 1]], g0, sem0).wait()
  pltpu.sync_copy(g0, acc.at[colv.at[_K - 1]], add=True)

  plsc.subcore_barrier()
  pltpu.sync_copy(acc.at[pl.ds(sid * _NPS, _NPS)],
                  out_hbm.at[cid, pl.ds(sid * _NPS, _NPS)])


_edge_call = functools.partial(
    pl.kernel,
    out_type=jax.ShapeDtypeStruct((_NC, _NPAD, _H), jnp.float32),
    mesh=_mesh,
    scratch_types=[
        pltpu.VMEM((_K, _C), jnp.int32),
        pltpu.VMEM((_K, _C), jnp.int32),
        pltpu.VMEM((_C, _H), jnp.float32),
        pltpu.VMEM((_C, _H), jnp.float32),
        pltpu.VMEM((_NPS, _H), jnp.float32),
        pltpu.VMEM_SHARED((_NPAD, _H), jnp.float32),
        pltpu.SemaphoreType.DMA,
        pltpu.SemaphoreType.DMA,
    ],
    compiler_params=pltpu.CompilerParams(use_tc_tiling_on_sc=False),
)(_edge_body)


def _tc1_body(degp_ref, x_ref, w1_ref, dis_ref, z1_ref):
  ones = jnp.ones((_NW, _H), jnp.float32)
  deg = lax.dot_general(degp_ref[...], ones, (((0,), (0,)), ((), ())),
                        preferred_element_type=jnp.float32) + 1.0
  dis = lax.rsqrt(deg)
  dis_ref[...] = dis
  z1_ref[...] = dis * jnp.dot(x_ref[...], w1_ref[...],
                              preferred_element_type=jnp.float32)


def _tc_mid_body(accp_ref, z_ref, dis_ref, b_ref, w_ref, zo_ref):
  dis = dis_ref[...]
  s = accp_ref[0] + accp_ref[1] + z_ref[...]
  h = jnp.maximum(dis * s + b_ref[...], 0.0)
  zo_ref[...] = dis * jnp.dot(h, w_ref[...], preferred_element_type=jnp.float32)


def _tc_fin_body(accp_ref, z_ref, dis_ref, b3_ref, batch_ref, wl_ref, bl_ref,
                 out_ref):
  o3 = dis_ref[...] * (accp_ref[0] + accp_ref[1] + z_ref[...]) + b3_ref[...]
  gid = lax.broadcasted_iota(jnp.int32, (_G, _NPAD), 0)
  m = (gid == batch_ref[...]).astype(jnp.float32)
  sums = jnp.dot(m, o3, preferred_element_type=jnp.float32)
  counts = jnp.sum(m, axis=1, keepdims=True)
  pooled = sums / jnp.maximum(counts, 1.0)
  out_ref[...] = jnp.dot(pooled, wl_ref[...],
                         preferred_element_type=jnp.float32) + bl_ref[...]


def kernel(x, edge_index, batch, W1, b1, W2, b2, W3, b3, Wl, bl):
  row = edge_index[0]
  col = edge_index[1]
  pad = _EP - _E
  rowp = jnp.concatenate(
      [row, jnp.zeros((pad,), jnp.int32)]).reshape(_NW, _K, _C)
  colp = jnp.concatenate(
      [col, jnp.full((pad,), _N, jnp.int32)]).reshape(_NW, _K, _C)
  xp = jnp.pad(x.astype(jnp.float32), ((0, _NPAD - _N), (0, 0)))
  batchp = jnp.pad(batch, (0, _NPAD - _N),
                   constant_values=_G).reshape(1, _NPAD)
  b1r = b1.reshape(1, _H)
  b2r = b2.reshape(1, _H)
  b3r = b3.reshape(1, _H)
  blr = bl.reshape(1, _DOUT)

  degp = _deg_call(colp)

  dis, z1 = pl.pallas_call(
      _tc1_body,
      out_shape=(jax.ShapeDtypeStruct((_NPAD, _H), jnp.float32),
                 jax.ShapeDtypeStruct((_NPAD, _H), jnp.float32)),
  )(degp, xp, W1)

  acc1 = _edge_call(z1, rowp, colp)

  z2 = pl.pallas_call(
      _tc_mid_body,
      out_shape=jax.ShapeDtypeStruct((_NPAD, _H), jnp.float32),
  )(acc1, z1, dis, b1r, W2)

  acc2 = _edge_call(z2, rowp, colp)

  z3 = pl.pallas_call(
      _tc_mid_body,
      out_shape=jax.ShapeDtypeStruct((_NPAD, _H), jnp.float32),
  )(acc2, z2, dis, b2r, W3)

  acc3 = _edge_call(z3, rowp, colp)

  out = pl.pallas_call(
      _tc_fin_body,
      out_shape=jax.ShapeDtypeStruct((_G, _DOUT), jnp.float32),
  )(acc3, z3, dis, b3r, batchp, Wl, blr)

  return out
